# Initial kernel scaffold; baseline (speedup 1.0000x reference)
import jax
import jax.numpy as jnp
from jax.experimental import pallas as pl


def kernel(node_features, edge_index, edge_attr_tensor, node_attr_scalar_raw, W1_ss, W1_vv, W1_sv, W1_vs, Wlm_s, Wlm_v, W2_ss, W2_vv, W2_sv, W2_vs, Wlu_s, Wlu_v):
    raise NotImplementedError("write your pallas kernel here")


# speedup vs baseline: 2.0434x; 2.0434x over previous
import functools
import numpy as np
import jax
import jax.numpy as jnp
from jax.experimental import pallas as pl
from jax.experimental.pallas import tpu as pltpu

MUL_S = 16
MUL_V = 8
DF = MUL_S + 3 * MUL_V


def _kron(a, b):
    return jnp.einsum('uw,ij->uiwj', a, b).reshape(
        a.shape[0] * b.shape[0], a.shape[1] * b.shape[1])


def _edge_tables(W1_ss, W1_vv, W1_sv, W1_vs, Wlm_s, Wlm_v):
    a = 1.0 / np.sqrt(MUL_S + MUL_V)
    i3 = jnp.eye(3, dtype=jnp.float32)
    z = jnp.zeros((DF, DF), dtype=jnp.float32)
    t0 = z.at[:16, :16].set(Wlm_s / np.sqrt(MUL_S))
    t0 = t0.at[16:, 16:].set(_kron(Wlm_v, i3) / np.sqrt(MUL_V))
    t1 = z.at[:16, :16].set(a * W1_ss[:, 0, :])
    t1 = t1.at[16:, 16:].set(a * _kron(W1_vs[:, 0, :], i3))
    ts = [t0, t1]
    for j in range(3):
        ej = i3[:, j:j + 1]
        tj = z.at[16:, :16].set(a / np.sqrt(3.0) * _kron(W1_vv[:, 0, :], ej))
        tj = tj.at[:16, 16:].set(a * _kron(W1_sv[:, 0, :], ej.T))
        ts.append(tj)
    return jnp.concatenate(ts, axis=1)


def _node_tables(W2_ss, W2_vv, W2_sv, W2_vs, Wlu_s, Wlu_v):
    a_s2 = 1.0 / np.sqrt(MUL_S * MUL_S + MUL_V * MUL_V)
    a_v2 = 1.0 / np.sqrt(2 * MUL_S * MUL_V)
    i3 = jnp.eye(3, dtype=jnp.float32)
    t2 = jnp.zeros((DF, DF, DF), dtype=jnp.float32)
    t2 = t2.at[:16, :16, :16].set(a_s2 * W2_ss)
    blk = jnp.einsum('uvw,ij->uivjw', W2_vv, i3).reshape(24, 24, 16)
    t2 = t2.at[16:, 16:, :16].set(blk * (a_s2 / np.sqrt(3.0)))
    blk = jnp.einsum('uvw,ij->uviwj', W2_sv, i3).reshape(16, 24, 24)
    t2 = t2.at[:16, 16:, 16:].set(a_v2 * blk)
    blk = jnp.einsum('uvw,ij->uivwj', W2_vs, i3).reshape(24, 16, 24)
    t2 = t2.at[16:, :16, 16:].set(a_v2 * blk)
    tsc = jnp.eye(DF, dtype=jnp.float32)
    tsc = tsc.at[:16, :16].add(Wlu_s / np.sqrt(MUL_S))
    tsc = tsc.at[16:, 16:].add(_kron(Wlu_v, i3) / np.sqrt(MUL_V))
    return t2.reshape(DF, DF * DF), tsc


def _msg_body(xg_ref, ea_ref, t_ref, o_ref):
    xg = xg_ref[...]
    y = jnp.dot(xg, t_ref[...], preferred_element_type=jnp.float32)
    ea = ea_ref[...]
    m = (y[:, 0:40]
         + ea[:, 3:4] * y[:, 40:80]
         + ea[:, 0:1] * y[:, 80:120]
         + ea[:, 1:2] * y[:, 120:160]
         + ea[:, 2:3] * y[:, 160:200])
    o_ref[...] = m


def _upd_body(x_ref, a_ref, t2_ref, tsc_ref, o_ref):
    x = x_ref[...]
    a = a_ref[...]
    acc = jnp.dot(x, tsc_ref[...], preferred_element_type=jnp.float32)
    for j in range(5):
        rc = jnp.dot(x, t2_ref[:, 320 * j:320 * (j + 1)],
                     preferred_element_type=jnp.float32)
        for gi in range(8):
            g = 8 * j + gi
            acc += a[:, g:g + 1] * rc[:, 40 * gi:40 * (gi + 1)]
    o_ref[...] = acc


def _pick_block(n, cap):
    for b in range(cap, 7, -8):
        if n % b == 0:
            return b
    return None


def _messages(xg, ea, tcat):
    e = xg.shape[0]
    be = _pick_block(e, 4096) or 4096
    ep = ((e + be - 1) // be) * be
    if ep != e:
        xg = jnp.pad(xg, ((0, ep - e), (0, 0)))
        ea = jnp.pad(ea, ((0, ep - e), (0, 0)))
    out = pl.pallas_call(
        _msg_body,
        grid=(ep // be,),
        in_specs=[
            pl.BlockSpec((be, DF), lambda i: (i, 0)),
            pl.BlockSpec((be, 4), lambda i: (i, 0)),
            pl.BlockSpec((DF, 200), lambda i: (0, 0)),
        ],
        out_specs=pl.BlockSpec((be, DF), lambda i: (i, 0)),
        out_shape=jax.ShapeDtypeStruct((ep, DF), jnp.float32),
    )(xg, ea, tcat)
    return out[:e] if ep != e else out


def _update(x, agg, t2r, tsc):
    n = x.shape[0]
    bn = _pick_block(n, 1024) or 1024
    npad = ((n + bn - 1) // bn) * bn
    if npad != n:
        x = jnp.pad(x, ((0, npad - n), (0, 0)))
        agg = jnp.pad(agg, ((0, npad - n), (0, 0)))
    out = pl.pallas_call(
        _upd_body,
        grid=(npad // bn,),
        in_specs=[
            pl.BlockSpec((bn, DF), lambda i: (i, 0)),
            pl.BlockSpec((bn, DF), lambda i: (i, 0)),
            pl.BlockSpec((DF, DF * DF), lambda i: (0, 0)),
            pl.BlockSpec((DF, DF), lambda i: (0, 0)),
        ],
        out_specs=pl.BlockSpec((bn, DF), lambda i: (i, 0)),
        out_shape=jax.ShapeDtypeStruct((npad, DF), jnp.float32),
    )(x, agg, t2r, tsc)
    return out[:n] if npad != n else out


def kernel(node_features, edge_index, edge_attr_tensor, node_attr_scalar_raw,
           W1_ss, W1_vv, W1_sv, W1_vs, Wlm_s, Wlm_v,
           W2_ss, W2_vv, W2_sv, W2_vs, Wlu_s, Wlu_v):
    del node_attr_scalar_raw
    row = edge_index[0]
    col = edge_index[1]
    tcat = _edge_tables(W1_ss, W1_vv, W1_sv, W1_vs, Wlm_s, Wlm_v)
    t2r, tsc = _node_tables(W2_ss, W2_vv, W2_sv, W2_vs, Wlu_s, Wlu_v)
    xg = jnp.take(node_features, col, axis=0)
    messages = _messages(xg, edge_attr_tensor, tcat)
    agg = jax.ops.segment_sum(messages, row,
                              num_segments=node_features.shape[0])
    return _update(node_features, agg, t2r, tsc)


# speedup vs baseline: 2.7509x; 1.3462x over previous
import functools
import numpy as np
import jax
import jax.numpy as jnp
from jax import lax
from jax.experimental import pallas as pl
from jax.experimental.pallas import tpu as pltpu
from jax.experimental.pallas import tpu_sc as plsc

MUL_S = 16
MUL_V = 8
DF = MUL_S + 3 * MUL_V


def _kron(a, b):
    return jnp.einsum('uw,ij->uiwj', a, b).reshape(
        a.shape[0] * b.shape[0], a.shape[1] * b.shape[1])


def _edge_tables(W1_ss, W1_vv, W1_sv, W1_vs, Wlm_s, Wlm_v):
    a = 1.0 / np.sqrt(MUL_S + MUL_V)
    i3 = jnp.eye(3, dtype=jnp.float32)
    z = jnp.zeros((DF, DF), dtype=jnp.float32)
    t0 = z.at[:16, :16].set(Wlm_s / np.sqrt(MUL_S))
    t0 = t0.at[16:, 16:].set(_kron(Wlm_v, i3) / np.sqrt(MUL_V))
    t1 = z.at[:16, :16].set(a * W1_ss[:, 0, :])
    t1 = t1.at[16:, 16:].set(a * _kron(W1_vs[:, 0, :], i3))
    ts = [t0, t1]
    for j in range(3):
        ej = i3[:, j:j + 1]
        tj = z.at[16:, :16].set(a / np.sqrt(3.0) * _kron(W1_vv[:, 0, :], ej))
        tj = tj.at[:16, 16:].set(a * _kron(W1_sv[:, 0, :], ej.T))
        ts.append(tj)
    return jnp.concatenate(ts, axis=1)


def _node_tables(W2_ss, W2_vv, W2_sv, W2_vs, Wlu_s, Wlu_v):
    a_s2 = 1.0 / np.sqrt(MUL_S * MUL_S + MUL_V * MUL_V)
    a_v2 = 1.0 / np.sqrt(2 * MUL_S * MUL_V)
    i3 = jnp.eye(3, dtype=jnp.float32)
    t2 = jnp.zeros((DF, DF, DF), dtype=jnp.float32)
    t2 = t2.at[:16, :16, :16].set(a_s2 * W2_ss)
    blk = jnp.einsum('uvw,ij->uivjw', W2_vv, i3).reshape(24, 24, 16)
    t2 = t2.at[16:, 16:, :16].set(blk * (a_s2 / np.sqrt(3.0)))
    blk = jnp.einsum('uvw,ij->uviwj', W2_sv, i3).reshape(16, 24, 24)
    t2 = t2.at[:16, 16:, 16:].set(a_v2 * blk)
    blk = jnp.einsum('uvw,ij->uivwj', W2_vs, i3).reshape(24, 16, 24)
    t2 = t2.at[16:, :16, 16:].set(a_v2 * blk)
    tsc = jnp.eye(DF, dtype=jnp.float32)
    tsc = tsc.at[:16, :16].add(Wlu_s / np.sqrt(MUL_S))
    tsc = tsc.at[16:, 16:].add(_kron(Wlu_v, i3) / np.sqrt(MUL_V))
    return t2.reshape(DF, DF * DF), tsc


def _msg_body(xg_ref, ea_ref, t_ref, o_ref):
    xg = xg_ref[...]
    y = jnp.dot(xg, t_ref[...], preferred_element_type=jnp.float32)
    ea = ea_ref[...]
    m = (y[:, 0:40]
         + ea[:, 3:4] * y[:, 40:80]
         + ea[:, 0:1] * y[:, 80:120]
         + ea[:, 1:2] * y[:, 120:160]
         + ea[:, 2:3] * y[:, 160:200])
    o_ref[...] = m


def _upd_body(x_ref, a_ref, t2_ref, tsc_ref, o_ref):
    x = x_ref[...]
    a = a_ref[...]
    acc = jnp.dot(x, tsc_ref[...], preferred_element_type=jnp.float32)
    for j in range(5):
        rc = jnp.dot(x, t2_ref[:, 320 * j:320 * (j + 1)],
                     preferred_element_type=jnp.float32)
        for gi in range(8):
            g = 8 * j + gi
            acc += a[:, g:g + 1] * rc[:, 40 * gi:40 * (gi + 1)]
    o_ref[...] = acc


def _pick_block(n, cap):
    for b in range(cap, 7, -8):
        if n % b == 0:
            return b
    return None


def _messages(xg, ea, tcat):
    e = xg.shape[0]
    be = _pick_block(e, 4096) or 4096
    ep = ((e + be - 1) // be) * be
    if ep != e:
        xg = jnp.pad(xg, ((0, ep - e), (0, 0)))
        ea = jnp.pad(ea, ((0, ep - e), (0, 0)))
    out = pl.pallas_call(
        _msg_body,
        grid=(ep // be,),
        in_specs=[
            pl.BlockSpec((be, DF), lambda i: (i, 0)),
            pl.BlockSpec((be, 4), lambda i: (i, 0)),
            pl.BlockSpec((DF, 200), lambda i: (0, 0)),
        ],
        out_specs=pl.BlockSpec((be, DF), lambda i: (i, 0)),
        out_shape=jax.ShapeDtypeStruct((ep, DF), jnp.float32),
    )(xg, ea, tcat)
    return out[:e] if ep != e else out


def _update(x, agg, t2r, tsc):
    n = x.shape[0]
    bn = _pick_block(n, 1024) or 1024
    npad = ((n + bn - 1) // bn) * bn
    if npad != n:
        x = jnp.pad(x, ((0, npad - n), (0, 0)))
        agg = jnp.pad(agg, ((0, npad - n), (0, 0)))
    out = pl.pallas_call(
        _upd_body,
        grid=(npad // bn,),
        in_specs=[
            pl.BlockSpec((bn, DF), lambda i: (i, 0)),
            pl.BlockSpec((bn, DF), lambda i: (i, 0)),
            pl.BlockSpec((DF, DF * DF), lambda i: (0, 0)),
            pl.BlockSpec((DF, DF), lambda i: (0, 0)),
        ],
        out_specs=pl.BlockSpec((bn, DF), lambda i: (i, 0)),
        out_shape=jax.ShapeDtypeStruct((npad, DF), jnp.float32),
    )(x, agg, t2r, tsc)
    return out[:n] if npad != n else out


def _sc_gather(nodes, col):
    e = col.shape[0]
    d = nodes.shape[1]
    info = plsc.get_sparse_core_info()
    nc, ns = info.num_cores, info.num_subcores
    nw = nc * ns
    ch = 2000
    if e % (nw * ch) != 0:
        return jnp.take(nodes, col, axis=0)
    per_w = e // nw
    mesh = plsc.VectorSubcoreMesh(core_axis_name="c", subcore_axis_name="s")

    @functools.partial(
        pl.kernel,
        out_type=jax.ShapeDtypeStruct((e, d), jnp.float32),
        mesh=mesh,
        scratch_types=[
            pltpu.VMEM((ch,), jnp.int32),
            pltpu.VMEM((ch, d), jnp.float32),
            pltpu.SemaphoreType.DMA,
        ],
        compiler_params=pltpu.CompilerParams(use_tc_tiling_on_sc=False),
    )
    def gather_k(nodes_hbm, col_hbm, out_hbm, idx_v, rows_v, sem):
        wid = lax.axis_index("s") * nc + lax.axis_index("c")
        base0 = wid * per_w

        def body(i, carry):
            base = base0 + i * ch
            pltpu.sync_copy(col_hbm.at[pl.ds(base, ch)], idx_v)
            pltpu.async_copy(nodes_hbm.at[idx_v], rows_v, sem).wait()
            pltpu.sync_copy(rows_v, out_hbm.at[pl.ds(base, ch)])
            return carry

        lax.fori_loop(0, per_w // ch, body, 0)

    return gather_k(nodes, col)


def kernel(node_features, edge_index, edge_attr_tensor, node_attr_scalar_raw,
           W1_ss, W1_vv, W1_sv, W1_vs, Wlm_s, Wlm_v,
           W2_ss, W2_vv, W2_sv, W2_vs, Wlu_s, Wlu_v):
    del node_attr_scalar_raw
    row = edge_index[0]
    col = edge_index[1]
    tcat = _edge_tables(W1_ss, W1_vv, W1_sv, W1_vs, Wlm_s, Wlm_v)
    t2r, tsc = _node_tables(W2_ss, W2_vv, W2_sv, W2_vs, Wlu_s, Wlu_v)
    xg = _sc_gather(node_features, col)
    messages = _messages(xg, edge_attr_tensor, tcat)
    agg = jax.ops.segment_sum(messages, row,
                              num_segments=node_features.shape[0])
    return _update(node_features, agg, t2r, tsc)
